# Initial kernel scaffold; baseline (speedup 1.0000x reference)
import jax
import jax.numpy as jnp
from jax.experimental import pallas as pl


def kernel(I1, C1, C2, emb1, emb2, W1, b1, W2, b2):
    raise NotImplementedError("write your pallas kernel here")


# speedup vs baseline: 1.9300x; 1.9300x over previous
import functools

import jax
import jax.numpy as jnp
from jax import lax
from jax.experimental import pallas as pl
from jax.experimental.pallas import tpu as pltpu
from jax.experimental.pallas import tpu_sc as plsc

B = 16384
VOCAB = 1000
EMB = 16
ND = 13

NC = 2
NS = 16
NW = NC * NS
BPW = B // NW
CH = 128
NCH = BPW // CH

_mesh = plsc.VectorSubcoreMesh(core_axis_name="c", subcore_axis_name="s")


@functools.partial(
    pl.kernel,
    mesh=_mesh,
    compiler_params=pltpu.CompilerParams(use_tc_tiling_on_sc=False),
    out_type=(
        jax.ShapeDtypeStruct((B, EMB), jnp.float32),
        jax.ShapeDtypeStruct((B, EMB), jnp.float32),
    ),
    scratch_types=[
        pltpu.VMEM((NCH, CH), jnp.int32),
        pltpu.VMEM((NCH, CH), jnp.int32),
        pltpu.VMEM((BPW, EMB), jnp.float32),
        pltpu.VMEM((BPW, EMB), jnp.float32),
        pltpu.SemaphoreType.DMA,
        pltpu.SemaphoreType.DMA,
    ],
)
def _sc_gather(emb1_hbm, emb2_hbm, c1_hbm, c2_hbm, o1_hbm, o2_hbm,
               idx1, idx2, r1, r2, sem1, sem2):
    wid = lax.axis_index("s") * NC + lax.axis_index("c")
    base = wid * BPW
    pltpu.sync_copy(c1_hbm.at[wid], idx1)
    pltpu.sync_copy(c2_hbm.at[wid], idx2)
    cps = []
    for k in range(NCH):
        cps.append(pltpu.async_copy(
            emb1_hbm.at[idx1.at[k]], r1.at[pl.ds(k * CH, CH)], sem1))
        cps.append(pltpu.async_copy(
            emb2_hbm.at[idx2.at[k]], r2.at[pl.ds(k * CH, CH)], sem2))
    for cp in cps:
        cp.wait()
    pltpu.sync_copy(r1, o1_hbm.at[pl.ds(base, BPW)])
    pltpu.sync_copy(r2, o2_hbm.at[pl.ds(base, BPW)])


def _mlp_body(i1_ref, e1_ref, e2_ref, w1a_ref, w1b_ref, w1c_ref, b1_ref,
              w2_ref, b2_ref, o_ref):
    h = jnp.dot(i1_ref[...], w1a_ref[...], preferred_element_type=jnp.float32)
    h = h + jnp.dot(e1_ref[...], w1b_ref[...],
                    preferred_element_type=jnp.float32)
    h = h + jnp.dot(e2_ref[...], w1c_ref[...],
                    preferred_element_type=jnp.float32)
    h = jnp.maximum(h + b1_ref[...], 0.0)
    o_ref[...] = jnp.sum(h * w2_ref[...], axis=1, keepdims=True) + b2_ref[...]


BLK = 2048


def _mlp_call(I1, e1g, e2g, W1a, W1b, W1c, b1r, W2r, b2r):
    grid = (B // BLK,)
    return pl.pallas_call(
        _mlp_body,
        grid=grid,
        in_specs=[
            pl.BlockSpec((BLK, ND), lambda i: (i, 0)),
            pl.BlockSpec((BLK, EMB), lambda i: (i, 0)),
            pl.BlockSpec((BLK, EMB), lambda i: (i, 0)),
            pl.BlockSpec((ND, 16), lambda i: (0, 0)),
            pl.BlockSpec((EMB, 16), lambda i: (0, 0)),
            pl.BlockSpec((EMB, 16), lambda i: (0, 0)),
            pl.BlockSpec((1, 16), lambda i: (0, 0)),
            pl.BlockSpec((1, 16), lambda i: (0, 0)),
            pl.BlockSpec((1, 1), lambda i: (0, 0)),
        ],
        out_specs=pl.BlockSpec((BLK, 1), lambda i: (i, 0)),
        out_shape=jax.ShapeDtypeStruct((B, 1), jnp.float32),
    )(I1, e1g, e2g, W1a, W1b, W1c, b1r, W2r, b2r)


def kernel(I1, C1, C2, emb1, emb2, W1, b1, W2, b2):
    c1 = C1.astype(jnp.int32).reshape(NW, NCH, CH)
    c2 = C2.astype(jnp.int32).reshape(NW, NCH, CH)
    e1g, e2g = _sc_gather(emb1, emb2, c1, c2)
    return _mlp_call(
        I1, e1g, e2g,
        W1[:ND], W1[ND:ND + EMB], W1[ND + EMB:],
        b1.reshape(1, EMB), W2.reshape(1, EMB), b2.reshape(1, 1))


# speedup vs baseline: 1.9748x; 1.0233x over previous
import functools

import jax
import jax.numpy as jnp
from jax import lax
from jax.experimental import pallas as pl
from jax.experimental.pallas import tpu as pltpu
from jax.experimental.pallas import tpu_sc as plsc

B = 16384
VOCAB = 1000
EMB = 16
ND = 13

NC = 2
NS = 16
NW = NC * NS
BPW = B // NW
CH = 128
NCH = BPW // CH

_mesh = plsc.VectorSubcoreMesh(core_axis_name="c", subcore_axis_name="s")


@functools.partial(
    pl.kernel,
    mesh=_mesh,
    compiler_params=pltpu.CompilerParams(use_tc_tiling_on_sc=False),
    out_type=(
        jax.ShapeDtypeStruct((B, EMB), jnp.float32),
        jax.ShapeDtypeStruct((B, EMB), jnp.float32),
    ),
    scratch_types=[
        pltpu.VMEM((NCH, CH), jnp.int32),
        pltpu.VMEM((NCH, CH), jnp.int32),
        pltpu.VMEM((BPW, EMB), jnp.float32),
        pltpu.VMEM((BPW, EMB), jnp.float32),
        pltpu.SemaphoreType.DMA,
        pltpu.SemaphoreType.DMA,
    ],
)
def _sc_gather(emb1_hbm, emb2_hbm, c1_hbm, c2_hbm, o1_hbm, o2_hbm,
               idx1, idx2, r1, r2, sem1, sem2):
    wid = lax.axis_index("s") * NC + lax.axis_index("c")
    base = wid * BPW
    pltpu.sync_copy(c1_hbm.at[wid], idx1)
    pltpu.sync_copy(c2_hbm.at[wid], idx2)
    cps = []
    for k in range(NCH):
        cps.append(pltpu.async_copy(
            emb1_hbm.at[idx1.at[k]], r1.at[pl.ds(k * CH, CH)], sem1))
        cps.append(pltpu.async_copy(
            emb2_hbm.at[idx2.at[k]], r2.at[pl.ds(k * CH, CH)], sem2))
    for cp in cps:
        cp.wait()
    pltpu.sync_copy(r1, o1_hbm.at[pl.ds(base, BPW)])
    pltpu.sync_copy(r2, o2_hbm.at[pl.ds(base, BPW)])


def _mlp_body(i1_ref, e1_ref, e2_ref, w1_ref, b1_ref, w2_ref, b2_ref, o_ref):
    w1 = w1_ref[...]
    h = jnp.dot(i1_ref[...], w1[:ND, :], preferred_element_type=jnp.float32)
    h = h + jnp.dot(e1_ref[...], w1[ND:ND + EMB, :],
                    preferred_element_type=jnp.float32)
    h = h + jnp.dot(e2_ref[...], w1[ND + EMB:, :],
                    preferred_element_type=jnp.float32)
    h = jnp.maximum(h + b1_ref[...], 0.0)
    o_ref[...] = jnp.sum(h * w2_ref[...], axis=1, keepdims=True) + b2_ref[...]


BLK = 2048


def _mlp_call(I1, e1g, e2g, W1, b1r, W2r, b2r):
    grid = (B // BLK,)
    return pl.pallas_call(
        _mlp_body,
        grid=grid,
        in_specs=[
            pl.BlockSpec((BLK, ND), lambda i: (i, 0)),
            pl.BlockSpec((BLK, EMB), lambda i: (i, 0)),
            pl.BlockSpec((BLK, EMB), lambda i: (i, 0)),
            pl.BlockSpec((ND + 2 * EMB, 16), lambda i: (0, 0)),
            pl.BlockSpec((1, 16), lambda i: (0, 0)),
            pl.BlockSpec((1, 16), lambda i: (0, 0)),
            pl.BlockSpec((1, 1), lambda i: (0, 0)),
        ],
        out_specs=pl.BlockSpec((BLK, 1), lambda i: (i, 0)),
        out_shape=jax.ShapeDtypeStruct((B, 1), jnp.float32),
    )(I1, e1g, e2g, W1, b1r, W2r, b2r)


def kernel(I1, C1, C2, emb1, emb2, W1, b1, W2, b2):
    c1 = C1.astype(jnp.int32).reshape(NW, NCH, CH)
    c2 = C2.astype(jnp.int32).reshape(NW, NCH, CH)
    e1g, e2g = _sc_gather(emb1, emb2, c1, c2)
    return _mlp_call(I1, e1g, e2g, W1,
                     b1.reshape(1, EMB), W2.reshape(1, EMB), b2.reshape(1, 1))
